# baseline (speedup 1.0000x reference)
import functools

import jax
import jax.numpy as jnp
from jax import lax
from jax.experimental import pallas as pl
from jax.experimental.pallas import tpu as pltpu
from jax.experimental.pallas import tpu_sc as plsc

N = 10000
E = 320000
IN_DIM = 128
HID = 256
OUT_DIM = 128

NC = 2
NS = 16
NW = NC * NS
B = 128
NB = 80
GW = 16
EP = NB * NW * B
NPAD = 10240
RPT = NPAD // NS
BN = 1024
GRID = NPAD // BN

_mesh = plsc.VectorSubcoreMesh(
    core_axis_name="c", subcore_axis_name="s", num_cores=NC, num_subcores=NS)


def _deg_body(dstr, init2, ones1, deg_out, acc, idxv, onesv, dsem):
    cid = lax.axis_index("c")
    sid = lax.axis_index("s")
    wid = cid * NS + sid
    pltpu.sync_copy(dstr.at[wid], idxv)
    pltpu.sync_copy(ones1, onesv)
    pltpu.sync_copy(init2.at[cid], acc.at[pl.ds(sid * RPT, RPT)])
    plsc.subcore_barrier()

    @pl.loop(0, NB // 8)
    def _(jj):
        descs = [
            pltpu.async_copy(onesv, acc.at[idxv.at[jj * 8 + k]], dsem,
                             add=True)
            for k in range(8)
        ]
        for d in descs:
            d.wait()

    plsc.subcore_barrier()
    pltpu.sync_copy(acc.at[pl.ds(sid * RPT, RPT)],
                    deg_out.at[cid, pl.ds(sid * RPT, RPT)])


_deg_kernel = pl.kernel(
    _deg_body,
    out_type=jax.ShapeDtypeStruct((NC, NPAD, IN_DIM), jnp.float32),
    mesh=_mesh,
    scratch_types=[
        pltpu.VMEM_SHARED((NPAD, IN_DIM), jnp.float32),
        pltpu.VMEM((NB, B), jnp.int32),
        pltpu.VMEM((B, IN_DIM), jnp.float32),
        pltpu.SemaphoreType.DMA,
    ],
)


def _agg_body(gtabs, srcr, dstr, zeros, outs, acc, srcv, dstv,
              buf0, buf1, g0, g1, s0, s1):
    C = len(gtabs)
    cid = lax.axis_index("c")
    sid = lax.axis_index("s")
    wid = cid * NS + sid
    for c in range(C):
        gtab = gtabs[c]

        def gather(j, buf, sem):
            return pltpu.async_copy(gtab.at[srcv.at[j]], buf, sem)

        def wait_gather(j, buf, sem):
            pltpu.make_async_copy(gtab.at[srcv.at[j]], buf, sem).wait()

        pltpu.sync_copy(zeros, acc.at[pl.ds(sid * RPT, RPT)])
        plsc.subcore_barrier()

        @pl.loop(0, NB // GW)
        def _(g):
            pltpu.sync_copy(srcr.at[wid, pl.ds(g * GW, GW)], srcv)
            pltpu.sync_copy(dstr.at[wid, pl.ds(g * GW, GW)], dstv)
            gather(0, buf0, g0)
            gather(1, buf1, g1)

            @pl.loop(0, (GW - 2) // 2)
            def _(jj):
                j = jj * 2
                wait_gather(j, buf0, g0)
                sd0 = pltpu.async_copy(buf0, acc.at[dstv.at[j]], s0,
                                       add=True)
                wait_gather(j + 1, buf1, g1)
                sd1 = pltpu.async_copy(buf1, acc.at[dstv.at[j + 1]], s1,
                                       add=True)
                sd0.wait()
                gather(j + 2, buf0, g0)
                sd1.wait()
                gather(j + 3, buf1, g1)

            wait_gather(GW - 2, buf0, g0)
            sd0 = pltpu.async_copy(buf0, acc.at[dstv.at[GW - 2]], s0,
                                   add=True)
            wait_gather(GW - 1, buf1, g1)
            sd1 = pltpu.async_copy(buf1, acc.at[dstv.at[GW - 1]], s1,
                                   add=True)
            sd0.wait()
            sd1.wait()

        plsc.subcore_barrier()
        pltpu.sync_copy(acc.at[pl.ds(sid * RPT, RPT)],
                        outs[c].at[cid, pl.ds(sid * RPT, RPT)])
        plsc.subcore_barrier()


def _make_agg(C):
    def body(*refs):
        gtabs = refs[0:C]
        srcr, dstr, zeros = refs[C:C + 3]
        outs = refs[C + 3:2 * C + 3]
        acc, srcv, dstv, buf0, buf1, g0, g1, s0, s1 = refs[2 * C + 3:]
        _agg_body(gtabs, srcr, dstr, zeros, outs, acc, srcv, dstv,
                  buf0, buf1, g0, g1, s0, s1)

    return pl.kernel(
        body,
        out_type=[jax.ShapeDtypeStruct((NC, NPAD, IN_DIM), jnp.float32)] * C,
        mesh=_mesh,
        scratch_types=[
            pltpu.VMEM_SHARED((NPAD, IN_DIM), jnp.float32),
            pltpu.VMEM((GW, B), jnp.int32),
            pltpu.VMEM((GW, B), jnp.int32),
            pltpu.VMEM((B, IN_DIM), jnp.float32),
            pltpu.VMEM((B, IN_DIM), jnp.float32),
            pltpu.SemaphoreType.DMA,
            pltpu.SemaphoreType.DMA,
            pltpu.SemaphoreType.DMA,
            pltpu.SemaphoreType.DMA,
        ],
    )


_agg1 = _make_agg(1)
_agg2 = _make_agg(2)


def _dinv_body(deg_ref, x_ref, dinv_ref, gx_ref):
    dsum = deg_ref[0, :, 0:1] + deg_ref[1, :, 0:1]
    dinv = lax.rsqrt(jnp.maximum(dsum, 1.0))
    dinv_ref[...] = jnp.broadcast_to(dinv, (BN, 16))
    gx_ref[...] = x_ref[...] * dinv


def _dinv_call(degp, xp):
    return pl.pallas_call(
        _dinv_body,
        grid=(GRID,),
        in_specs=[
            pl.BlockSpec((NC, BN, IN_DIM), lambda i: (0, i, 0)),
            pl.BlockSpec((BN, IN_DIM), lambda i: (i, 0)),
        ],
        out_specs=[
            pl.BlockSpec((BN, 16), lambda i: (i, 0)),
            pl.BlockSpec((BN, IN_DIM), lambda i: (i, 0)),
        ],
        out_shape=[
            jax.ShapeDtypeStruct((NPAD, 16), jnp.float32),
            jax.ShapeDtypeStruct((NPAD, IN_DIM), jnp.float32),
        ],
    )(degp, xp)


def _mm1_body(p_ref, g_ref, dinv_ref, w_ref, b_ref, h_ref, ga_ref, gb_ref):
    dinv = dinv_ref[:, 0:1]
    s = (p_ref[0] + p_ref[1] + g_ref[...]) * dinv
    h = jnp.maximum(jnp.dot(s, w_ref[...],
                            preferred_element_type=jnp.float32) + b_ref[...], 0.0)
    h_ref[...] = h
    ga_ref[...] = h[:, :IN_DIM] * dinv
    gb_ref[...] = h[:, IN_DIM:] * dinv


def _mm1_call(p1, gx, dinv8, W1, b1):
    return pl.pallas_call(
        _mm1_body,
        grid=(GRID,),
        in_specs=[
            pl.BlockSpec((NC, BN, IN_DIM), lambda i: (0, i, 0)),
            pl.BlockSpec((BN, IN_DIM), lambda i: (i, 0)),
            pl.BlockSpec((BN, 16), lambda i: (i, 0)),
            pl.BlockSpec((IN_DIM, HID), lambda i: (0, 0)),
            pl.BlockSpec((1, HID), lambda i: (0, 0)),
        ],
        out_specs=[
            pl.BlockSpec((BN, HID), lambda i: (i, 0)),
            pl.BlockSpec((BN, IN_DIM), lambda i: (i, 0)),
            pl.BlockSpec((BN, IN_DIM), lambda i: (i, 0)),
        ],
        out_shape=[
            jax.ShapeDtypeStruct((NPAD, HID), jnp.float32),
            jax.ShapeDtypeStruct((NPAD, IN_DIM), jnp.float32),
            jax.ShapeDtypeStruct((NPAD, IN_DIM), jnp.float32),
        ],
    )(p1, gx, dinv8, W1, b1)


def _mm2_body(pa_ref, pb_ref, ga_ref, gb_ref, dinv_ref, w2_ref, b2_ref,
              w3_ref, h_ref, g3_ref):
    dinv = dinv_ref[:, 0:1]
    s0 = (pa_ref[0] + pa_ref[1] + ga_ref[...]) * dinv
    s1 = (pb_ref[0] + pb_ref[1] + gb_ref[...]) * dinv
    w2 = w2_ref[...]
    h = jnp.dot(s0, w2[:IN_DIM], preferred_element_type=jnp.float32)
    h = h + jnp.dot(s1, w2[IN_DIM:], preferred_element_type=jnp.float32)
    h = jnp.maximum(h + b2_ref[...], 0.0)
    h_ref[...] = h
    hd = h * dinv
    w3 = w3_ref[...]
    g3 = jnp.dot(hd[:, :IN_DIM], w3[:IN_DIM], preferred_element_type=jnp.float32)
    g3 = g3 + jnp.dot(hd[:, IN_DIM:], w3[IN_DIM:], preferred_element_type=jnp.float32)
    g3_ref[...] = g3


def _mm2_call(p2a, p2b, g1a, g1b, dinv8, W2, b2, W3):
    return pl.pallas_call(
        _mm2_body,
        grid=(GRID,),
        in_specs=[
            pl.BlockSpec((NC, BN, IN_DIM), lambda i: (0, i, 0)),
            pl.BlockSpec((NC, BN, IN_DIM), lambda i: (0, i, 0)),
            pl.BlockSpec((BN, IN_DIM), lambda i: (i, 0)),
            pl.BlockSpec((BN, IN_DIM), lambda i: (i, 0)),
            pl.BlockSpec((BN, 16), lambda i: (i, 0)),
            pl.BlockSpec((HID, HID), lambda i: (0, 0)),
            pl.BlockSpec((1, HID), lambda i: (0, 0)),
            pl.BlockSpec((HID, OUT_DIM), lambda i: (0, 0)),
        ],
        out_specs=[
            pl.BlockSpec((BN, HID), lambda i: (i, 0)),
            pl.BlockSpec((BN, OUT_DIM), lambda i: (i, 0)),
        ],
        out_shape=[
            jax.ShapeDtypeStruct((NPAD, HID), jnp.float32),
            jax.ShapeDtypeStruct((NPAD, OUT_DIM), jnp.float32),
        ],
    )(p2a, p2b, g1a, g1b, dinv8, W2, b2, W3)


def _fin_body(p_ref, g_ref, dinv_ref, b_ref, h_ref):
    dinv = dinv_ref[:, 0:1]
    h_ref[...] = (p_ref[0] + p_ref[1] + g_ref[...]) * dinv + b_ref[...]


def _fin_call(p3, g3, dinv8, b3):
    return pl.pallas_call(
        _fin_body,
        grid=(GRID,),
        in_specs=[
            pl.BlockSpec((NC, BN, OUT_DIM), lambda i: (0, i, 0)),
            pl.BlockSpec((BN, OUT_DIM), lambda i: (i, 0)),
            pl.BlockSpec((BN, 16), lambda i: (i, 0)),
            pl.BlockSpec((1, OUT_DIM), lambda i: (0, 0)),
        ],
        out_specs=pl.BlockSpec((BN, OUT_DIM), lambda i: (i, 0)),
        out_shape=jax.ShapeDtypeStruct((NPAD, OUT_DIM), jnp.float32),
    )(p3, g3, dinv8, b3)


def kernel(x, edge_index, W1, b1, W2, b2, W3, b3):
    f32 = jnp.float32
    pad = jnp.arange(EP - E, dtype=jnp.int32)
    src = jnp.concatenate(
        [edge_index[0], pad % N]).reshape(NW, NB, B)
    dst = jnp.concatenate(
        [edge_index[1], N + pad % (NPAD - N)]).reshape(NW, NB, B)
    xp = jnp.pad(x, ((0, NPAD - N), (0, 0)))

    init2 = jnp.stack([jnp.ones((RPT, IN_DIM), f32), jnp.zeros((RPT, IN_DIM), f32)])
    ones1 = jnp.ones((B, IN_DIM), f32)
    zeros = jnp.zeros((RPT, IN_DIM), f32)

    degp = _deg_kernel(dst, init2, ones1)
    dinv8, gx = _dinv_call(degp, xp)

    (p1,) = _agg1(gx, src, dst, zeros)
    h1p, g1a, g1b = _mm1_call(p1, gx, dinv8, W1, b1.reshape(1, HID))

    p2a, p2b = _agg2(g1a, g1b, src, dst, zeros)
    h2p, g3 = _mm2_call(p2a, p2b, g1a, g1b, dinv8, W2,
                        b2.reshape(1, HID), W3)

    (p3,) = _agg1(g3, src, dst, zeros)
    h3p = _fin_call(p3, g3, dinv8, b3.reshape(1, OUT_DIM))

    h1, h2, h3 = h1p[:N], h2p[:N], h3p[:N]
    return (h3, (h1, h2, h3))


# speedup vs baseline: 1.2577x; 1.2577x over previous
import functools

import jax
import jax.numpy as jnp
from jax import lax
from jax.experimental import pallas as pl
from jax.experimental.pallas import tpu as pltpu
from jax.experimental.pallas import tpu_sc as plsc

N = 10000
E = 320000
IN_DIM = 128
HID = 256
OUT_DIM = 128

NC = 2
NS = 16
NW = NC * NS
B = 128
NB = 80
GW = 40
EP = NB * NW * B
NPAD = 10240
RPT = NPAD // NS
BN = 1024
GRID = NPAD // BN

_mesh = plsc.VectorSubcoreMesh(
    core_axis_name="c", subcore_axis_name="s", num_cores=NC, num_subcores=NS)


def _deg_body(dstr, init2, ones1, deg_out, acc, idxv, onesv, dsem):
    cid = lax.axis_index("c")
    sid = lax.axis_index("s")
    wid = cid * NS + sid
    pltpu.sync_copy(dstr.at[wid], idxv)
    pltpu.sync_copy(ones1, onesv)
    pltpu.sync_copy(init2.at[cid], acc.at[pl.ds(sid * RPT, RPT)])
    plsc.subcore_barrier()

    @pl.loop(0, NB // 8)
    def _(jj):
        descs = [
            pltpu.async_copy(onesv, acc.at[idxv.at[jj * 8 + k]], dsem,
                             add=True)
            for k in range(8)
        ]
        for d in descs:
            d.wait()

    plsc.subcore_barrier()
    pltpu.sync_copy(acc.at[pl.ds(sid * RPT, RPT)],
                    deg_out.at[cid, pl.ds(sid * RPT, RPT)])


_deg_kernel = pl.kernel(
    _deg_body,
    out_type=jax.ShapeDtypeStruct((NC, NPAD, IN_DIM), jnp.float32),
    mesh=_mesh,
    scratch_types=[
        pltpu.VMEM_SHARED((NPAD, IN_DIM), jnp.float32),
        pltpu.VMEM((NB, B), jnp.int32),
        pltpu.VMEM((B, IN_DIM), jnp.float32),
        pltpu.SemaphoreType.DMA,
    ],
)


def _agg_body(gtabs, srcr, dstr, zeros, outs, acc, srcv, dstv,
              buf0, buf1, g0, g1, s0, s1):
    C = len(gtabs)
    cid = lax.axis_index("c")
    sid = lax.axis_index("s")
    wid = cid * NS + sid
    for c in range(C):
        gtab = gtabs[c]

        def gather(j, buf, sem):
            return pltpu.async_copy(gtab.at[srcv.at[j]], buf, sem)

        def wait_gather(j, buf, sem):
            pltpu.make_async_copy(gtab.at[srcv.at[j]], buf, sem).wait()

        pltpu.sync_copy(zeros, acc.at[pl.ds(sid * RPT, RPT)])
        plsc.subcore_barrier()

        @pl.loop(0, NB // GW)
        def _(g):
            pltpu.sync_copy(srcr.at[wid, pl.ds(g * GW, GW)], srcv)
            pltpu.sync_copy(dstr.at[wid, pl.ds(g * GW, GW)], dstv)
            gather(0, buf0, g0)
            gather(1, buf1, g1)

            @pl.loop(0, (GW - 2) // 2)
            def _(jj):
                j = jj * 2
                wait_gather(j, buf0, g0)
                pltpu.async_copy(buf0, acc.at[dstv.at[j]], s0,
                                 add=True).wait()
                gather(j + 2, buf0, g0)
                wait_gather(j + 1, buf1, g1)
                pltpu.async_copy(buf1, acc.at[dstv.at[j + 1]], s1,
                                 add=True).wait()
                gather(j + 3, buf1, g1)

            wait_gather(GW - 2, buf0, g0)
            pltpu.async_copy(buf0, acc.at[dstv.at[GW - 2]], s0,
                             add=True).wait()
            wait_gather(GW - 1, buf1, g1)
            pltpu.async_copy(buf1, acc.at[dstv.at[GW - 1]], s1,
                             add=True).wait()

        plsc.subcore_barrier()
        pltpu.sync_copy(acc.at[pl.ds(sid * RPT, RPT)],
                        outs[c].at[cid, pl.ds(sid * RPT, RPT)])
        plsc.subcore_barrier()


def _make_agg(C):
    def body(*refs):
        gtabs = refs[0:C]
        srcr, dstr, zeros = refs[C:C + 3]
        outs = refs[C + 3:2 * C + 3]
        acc, srcv, dstv, buf0, buf1, g0, g1, s0, s1 = refs[2 * C + 3:]
        _agg_body(gtabs, srcr, dstr, zeros, outs, acc, srcv, dstv,
                  buf0, buf1, g0, g1, s0, s1)

    return pl.kernel(
        body,
        out_type=[jax.ShapeDtypeStruct((NC, NPAD, IN_DIM), jnp.float32)] * C,
        mesh=_mesh,
        scratch_types=[
            pltpu.VMEM_SHARED((NPAD, IN_DIM), jnp.float32),
            pltpu.VMEM((GW, B), jnp.int32),
            pltpu.VMEM((GW, B), jnp.int32),
            pltpu.VMEM((B, IN_DIM), jnp.float32),
            pltpu.VMEM((B, IN_DIM), jnp.float32),
            pltpu.SemaphoreType.DMA,
            pltpu.SemaphoreType.DMA,
            pltpu.SemaphoreType.DMA,
            pltpu.SemaphoreType.DMA,
        ],
    )


_agg1 = _make_agg(1)
_agg2 = _make_agg(2)


def _dinv_body(deg_ref, x_ref, dinv_ref, gx_ref):
    dsum = deg_ref[0, :, 0:1] + deg_ref[1, :, 0:1]
    dinv = lax.rsqrt(jnp.maximum(dsum, 1.0))
    dinv_ref[...] = jnp.broadcast_to(dinv, (BN, 16))
    gx_ref[...] = x_ref[...] * dinv


def _dinv_call(degp, xp):
    return pl.pallas_call(
        _dinv_body,
        grid=(GRID,),
        in_specs=[
            pl.BlockSpec((NC, BN, IN_DIM), lambda i: (0, i, 0)),
            pl.BlockSpec((BN, IN_DIM), lambda i: (i, 0)),
        ],
        out_specs=[
            pl.BlockSpec((BN, 16), lambda i: (i, 0)),
            pl.BlockSpec((BN, IN_DIM), lambda i: (i, 0)),
        ],
        out_shape=[
            jax.ShapeDtypeStruct((NPAD, 16), jnp.float32),
            jax.ShapeDtypeStruct((NPAD, IN_DIM), jnp.float32),
        ],
    )(degp, xp)


def _mm1_body(p_ref, g_ref, dinv_ref, w_ref, b_ref, h_ref, ga_ref, gb_ref):
    dinv = dinv_ref[:, 0:1]
    s = (p_ref[0] + p_ref[1] + g_ref[...]) * dinv
    h = jnp.maximum(jnp.dot(s, w_ref[...],
                            preferred_element_type=jnp.float32) + b_ref[...], 0.0)
    h_ref[...] = h
    ga_ref[...] = h[:, :IN_DIM] * dinv
    gb_ref[...] = h[:, IN_DIM:] * dinv


def _mm1_call(p1, gx, dinv8, W1, b1):
    return pl.pallas_call(
        _mm1_body,
        grid=(GRID,),
        in_specs=[
            pl.BlockSpec((NC, BN, IN_DIM), lambda i: (0, i, 0)),
            pl.BlockSpec((BN, IN_DIM), lambda i: (i, 0)),
            pl.BlockSpec((BN, 16), lambda i: (i, 0)),
            pl.BlockSpec((IN_DIM, HID), lambda i: (0, 0)),
            pl.BlockSpec((1, HID), lambda i: (0, 0)),
        ],
        out_specs=[
            pl.BlockSpec((BN, HID), lambda i: (i, 0)),
            pl.BlockSpec((BN, IN_DIM), lambda i: (i, 0)),
            pl.BlockSpec((BN, IN_DIM), lambda i: (i, 0)),
        ],
        out_shape=[
            jax.ShapeDtypeStruct((NPAD, HID), jnp.float32),
            jax.ShapeDtypeStruct((NPAD, IN_DIM), jnp.float32),
            jax.ShapeDtypeStruct((NPAD, IN_DIM), jnp.float32),
        ],
    )(p1, gx, dinv8, W1, b1)


def _mm2_body(pa_ref, pb_ref, ga_ref, gb_ref, dinv_ref, w2_ref, b2_ref,
              w3_ref, h_ref, g3_ref):
    dinv = dinv_ref[:, 0:1]
    s0 = (pa_ref[0] + pa_ref[1] + ga_ref[...]) * dinv
    s1 = (pb_ref[0] + pb_ref[1] + gb_ref[...]) * dinv
    w2 = w2_ref[...]
    h = jnp.dot(s0, w2[:IN_DIM], preferred_element_type=jnp.float32)
    h = h + jnp.dot(s1, w2[IN_DIM:], preferred_element_type=jnp.float32)
    h = jnp.maximum(h + b2_ref[...], 0.0)
    h_ref[...] = h
    hd = h * dinv
    w3 = w3_ref[...]
    g3 = jnp.dot(hd[:, :IN_DIM], w3[:IN_DIM], preferred_element_type=jnp.float32)
    g3 = g3 + jnp.dot(hd[:, IN_DIM:], w3[IN_DIM:], preferred_element_type=jnp.float32)
    g3_ref[...] = g3


def _mm2_call(p2a, p2b, g1a, g1b, dinv8, W2, b2, W3):
    return pl.pallas_call(
        _mm2_body,
        grid=(GRID,),
        in_specs=[
            pl.BlockSpec((NC, BN, IN_DIM), lambda i: (0, i, 0)),
            pl.BlockSpec((NC, BN, IN_DIM), lambda i: (0, i, 0)),
            pl.BlockSpec((BN, IN_DIM), lambda i: (i, 0)),
            pl.BlockSpec((BN, IN_DIM), lambda i: (i, 0)),
            pl.BlockSpec((BN, 16), lambda i: (i, 0)),
            pl.BlockSpec((HID, HID), lambda i: (0, 0)),
            pl.BlockSpec((1, HID), lambda i: (0, 0)),
            pl.BlockSpec((HID, OUT_DIM), lambda i: (0, 0)),
        ],
        out_specs=[
            pl.BlockSpec((BN, HID), lambda i: (i, 0)),
            pl.BlockSpec((BN, OUT_DIM), lambda i: (i, 0)),
        ],
        out_shape=[
            jax.ShapeDtypeStruct((NPAD, HID), jnp.float32),
            jax.ShapeDtypeStruct((NPAD, OUT_DIM), jnp.float32),
        ],
    )(p2a, p2b, g1a, g1b, dinv8, W2, b2, W3)


def _fin_body(p_ref, g_ref, dinv_ref, b_ref, h_ref):
    dinv = dinv_ref[:, 0:1]
    h_ref[...] = (p_ref[0] + p_ref[1] + g_ref[...]) * dinv + b_ref[...]


def _fin_call(p3, g3, dinv8, b3):
    return pl.pallas_call(
        _fin_body,
        grid=(GRID,),
        in_specs=[
            pl.BlockSpec((NC, BN, OUT_DIM), lambda i: (0, i, 0)),
            pl.BlockSpec((BN, OUT_DIM), lambda i: (i, 0)),
            pl.BlockSpec((BN, 16), lambda i: (i, 0)),
            pl.BlockSpec((1, OUT_DIM), lambda i: (0, 0)),
        ],
        out_specs=pl.BlockSpec((BN, OUT_DIM), lambda i: (i, 0)),
        out_shape=jax.ShapeDtypeStruct((NPAD, OUT_DIM), jnp.float32),
    )(p3, g3, dinv8, b3)


def kernel(x, edge_index, W1, b1, W2, b2, W3, b3):
    f32 = jnp.float32
    pad = jnp.arange(EP - E, dtype=jnp.int32)
    src = jnp.concatenate(
        [edge_index[0], pad % N]).reshape(NW, NB, B)
    dst = jnp.concatenate(
        [edge_index[1], N + pad % (NPAD - N)]).reshape(NW, NB, B)
    xp = jnp.pad(x, ((0, NPAD - N), (0, 0)))

    init2 = jnp.stack([jnp.ones((RPT, IN_DIM), f32), jnp.zeros((RPT, IN_DIM), f32)])
    ones1 = jnp.ones((B, IN_DIM), f32)
    zeros = jnp.zeros((RPT, IN_DIM), f32)

    degp = _deg_kernel(dst, init2, ones1)
    dinv8, gx = _dinv_call(degp, xp)

    (p1,) = _agg1(gx, src, dst, zeros)
    h1p, g1a, g1b = _mm1_call(p1, gx, dinv8, W1, b1.reshape(1, HID))

    p2a, p2b = _agg2(g1a, g1b, src, dst, zeros)
    h2p, g3 = _mm2_call(p2a, p2b, g1a, g1b, dinv8, W2,
                        b2.reshape(1, HID), W3)

    (p3,) = _agg1(g3, src, dst, zeros)
    h3p = _fin_call(p3, g3, dinv8, b3.reshape(1, OUT_DIM))

    h1, h2, h3 = h1p[:N], h2p[:N], h3p[:N]
    return (h3, (h1, h2, h3))


# speedup vs baseline: 1.2602x; 1.0020x over previous
import jax
import jax.numpy as jnp
from jax import lax
from jax.experimental import pallas as pl
from jax.experimental.pallas import tpu as pltpu
from jax.experimental.pallas import tpu_sc as plsc

N = 10000
E = 320000
IN_DIM = 128
HID = 256
OUT_DIM = 128

NC = 2
NS = 16
NW = NC * NS
B = 128
NB = 80
GW = 40
EP = NB * NW * B
NPAD = 10240
RPT = NPAD // NS
BN = 1024
GRID = NPAD // BN

_mesh = plsc.VectorSubcoreMesh(
    core_axis_name="c", subcore_axis_name="s", num_cores=NC, num_subcores=NS)


def _deg_body(dstr, init2, ones1, deg_out, acc, idxv, onesv, dsem):
    cid = lax.axis_index("c")
    sid = lax.axis_index("s")
    wid = cid * NS + sid
    pltpu.sync_copy(dstr.at[wid], idxv)
    pltpu.sync_copy(ones1, onesv)
    pltpu.sync_copy(init2.at[cid], acc.at[pl.ds(sid * RPT, RPT)])
    plsc.subcore_barrier()

    @pl.loop(0, NB // 8)
    def _(jj):
        descs = [
            pltpu.async_copy(onesv, acc.at[idxv.at[jj * 8 + k]], dsem,
                             add=True)
            for k in range(8)
        ]
        for d in descs:
            d.wait()

    plsc.subcore_barrier()
    pltpu.sync_copy(acc.at[pl.ds(sid * RPT, RPT)],
                    deg_out.at[cid, pl.ds(sid * RPT, RPT)])


_deg_kernel = pl.kernel(
    _deg_body,
    out_type=jax.ShapeDtypeStruct((NC, NPAD, IN_DIM), jnp.float32),
    mesh=_mesh,
    scratch_types=[
        pltpu.VMEM_SHARED((NPAD, IN_DIM), jnp.float32),
        pltpu.VMEM((NB, B), jnp.int32),
        pltpu.VMEM((B, IN_DIM), jnp.float32),
        pltpu.SemaphoreType.DMA,
    ],
)


def _agg_body(gtabs, srcr, dstr, zeros, outs, acc, srcv, dstv,
              buf0, buf1, g0, g1, s0, s1):
    C = len(gtabs)
    cid = lax.axis_index("c")
    sid = lax.axis_index("s")
    wid = cid * NS + sid
    for c in range(C):
        gtab = gtabs[c]

        def gather(j, buf, sem):
            return pltpu.async_copy(gtab.at[srcv.at[j]], buf, sem)

        def wait_gather(j, buf, sem):
            pltpu.make_async_copy(gtab.at[srcv.at[j]], buf, sem).wait()

        pltpu.sync_copy(zeros, acc.at[pl.ds(sid * RPT, RPT)])
        plsc.subcore_barrier()

        @pl.loop(0, NB // GW)
        def _(g):
            pltpu.sync_copy(srcr.at[wid, pl.ds(g * GW, GW)], srcv)
            pltpu.sync_copy(dstr.at[wid, pl.ds(g * GW, GW)], dstv)
            gather(0, buf0, g0)
            gather(1, buf1, g1)

            @pl.loop(0, (GW - 2) // 2)
            def _(jj):
                j = jj * 2
                wait_gather(j, buf0, g0)
                pltpu.async_copy(buf0, acc.at[dstv.at[j]], s0,
                                 add=True).wait()
                gather(j + 2, buf0, g0)
                wait_gather(j + 1, buf1, g1)
                pltpu.async_copy(buf1, acc.at[dstv.at[j + 1]], s1,
                                 add=True).wait()
                gather(j + 3, buf1, g1)

            wait_gather(GW - 2, buf0, g0)
            pltpu.async_copy(buf0, acc.at[dstv.at[GW - 2]], s0,
                             add=True).wait()
            wait_gather(GW - 1, buf1, g1)
            pltpu.async_copy(buf1, acc.at[dstv.at[GW - 1]], s1,
                             add=True).wait()

        plsc.subcore_barrier()
        pltpu.sync_copy(acc.at[pl.ds(sid * RPT, RPT)],
                        outs[c].at[cid, pl.ds(sid * RPT, RPT)])
        plsc.subcore_barrier()


def _make_agg(C):
    def body(*refs):
        gtabs = refs[0:C]
        srcr, dstr, zeros = refs[C:C + 3]
        outs = refs[C + 3:2 * C + 3]
        acc, srcv, dstv, buf0, buf1, g0, g1, s0, s1 = refs[2 * C + 3:]
        _agg_body(gtabs, srcr, dstr, zeros, outs, acc, srcv, dstv,
                  buf0, buf1, g0, g1, s0, s1)

    return pl.kernel(
        body,
        out_type=[jax.ShapeDtypeStruct((NC, NPAD, IN_DIM), jnp.float32)] * C,
        mesh=_mesh,
        scratch_types=[
            pltpu.VMEM_SHARED((NPAD, IN_DIM), jnp.float32),
            pltpu.VMEM((GW, B), jnp.int32),
            pltpu.VMEM((GW, B), jnp.int32),
            pltpu.VMEM((B, IN_DIM), jnp.float32),
            pltpu.VMEM((B, IN_DIM), jnp.float32),
            pltpu.SemaphoreType.DMA,
            pltpu.SemaphoreType.DMA,
            pltpu.SemaphoreType.DMA,
            pltpu.SemaphoreType.DMA,
        ],
    )


_agg1 = _make_agg(1)
_agg2 = _make_agg(2)


def _dinv_body(deg_ref, x_ref, dinv_ref, gx_ref):
    dsum = deg_ref[0, :, 0:1] + deg_ref[1, :, 0:1]
    dinv = lax.rsqrt(jnp.maximum(dsum, 1.0))
    dinv_ref[...] = jnp.broadcast_to(dinv, (BN, 16))
    gx_ref[...] = x_ref[...] * dinv


def _dinv_call(degp, xp):
    return pl.pallas_call(
        _dinv_body,
        grid=(GRID,),
        in_specs=[
            pl.BlockSpec((NC, BN, IN_DIM), lambda i: (0, i, 0)),
            pl.BlockSpec((BN, IN_DIM), lambda i: (i, 0)),
        ],
        out_specs=[
            pl.BlockSpec((BN, 16), lambda i: (i, 0)),
            pl.BlockSpec((BN, IN_DIM), lambda i: (i, 0)),
        ],
        out_shape=[
            jax.ShapeDtypeStruct((NPAD, 16), jnp.float32),
            jax.ShapeDtypeStruct((NPAD, IN_DIM), jnp.float32),
        ],
    )(degp, xp)


def _mm1_body(p_ref, g_ref, dinv_ref, w_ref, b_ref, h_ref, ga_ref, gb_ref):
    dinv = dinv_ref[:, 0:1]
    s = (p_ref[0] + p_ref[1] + g_ref[...]) * dinv
    h = jnp.maximum(jnp.dot(s, w_ref[...],
                            preferred_element_type=jnp.float32) + b_ref[...], 0.0)
    h_ref[...] = h
    ga_ref[...] = h[:, :IN_DIM] * dinv
    gb_ref[...] = h[:, IN_DIM:] * dinv


def _mm1_call(p1, gx, dinv8, W1, b1):
    return pl.pallas_call(
        _mm1_body,
        grid=(GRID,),
        in_specs=[
            pl.BlockSpec((NC, BN, IN_DIM), lambda i: (0, i, 0)),
            pl.BlockSpec((BN, IN_DIM), lambda i: (i, 0)),
            pl.BlockSpec((BN, 16), lambda i: (i, 0)),
            pl.BlockSpec((IN_DIM, HID), lambda i: (0, 0)),
            pl.BlockSpec((1, HID), lambda i: (0, 0)),
        ],
        out_specs=[
            pl.BlockSpec((BN, HID), lambda i: (i, 0)),
            pl.BlockSpec((BN, IN_DIM), lambda i: (i, 0)),
            pl.BlockSpec((BN, IN_DIM), lambda i: (i, 0)),
        ],
        out_shape=[
            jax.ShapeDtypeStruct((NPAD, HID), jnp.float32),
            jax.ShapeDtypeStruct((NPAD, IN_DIM), jnp.float32),
            jax.ShapeDtypeStruct((NPAD, IN_DIM), jnp.float32),
        ],
    )(p1, gx, dinv8, W1, b1)


def _mm2_body(pa_ref, pb_ref, ga_ref, gb_ref, dinv_ref, w2_ref, b2_ref,
              w3_ref, h_ref, g3_ref):
    dinv = dinv_ref[:, 0:1]
    s0 = (pa_ref[0] + pa_ref[1] + ga_ref[...]) * dinv
    s1 = (pb_ref[0] + pb_ref[1] + gb_ref[...]) * dinv
    w2 = w2_ref[...]
    h = jnp.dot(s0, w2[:IN_DIM], preferred_element_type=jnp.float32)
    h = h + jnp.dot(s1, w2[IN_DIM:], preferred_element_type=jnp.float32)
    h = jnp.maximum(h + b2_ref[...], 0.0)
    h_ref[...] = h
    hd = h * dinv
    w3 = w3_ref[...]
    g3 = jnp.dot(hd[:, :IN_DIM], w3[:IN_DIM], preferred_element_type=jnp.float32)
    g3 = g3 + jnp.dot(hd[:, IN_DIM:], w3[IN_DIM:], preferred_element_type=jnp.float32)
    g3_ref[...] = g3


def _mm2_call(p2a, p2b, g1a, g1b, dinv8, W2, b2, W3):
    return pl.pallas_call(
        _mm2_body,
        grid=(GRID,),
        in_specs=[
            pl.BlockSpec((NC, BN, IN_DIM), lambda i: (0, i, 0)),
            pl.BlockSpec((NC, BN, IN_DIM), lambda i: (0, i, 0)),
            pl.BlockSpec((BN, IN_DIM), lambda i: (i, 0)),
            pl.BlockSpec((BN, IN_DIM), lambda i: (i, 0)),
            pl.BlockSpec((BN, 16), lambda i: (i, 0)),
            pl.BlockSpec((HID, HID), lambda i: (0, 0)),
            pl.BlockSpec((1, HID), lambda i: (0, 0)),
            pl.BlockSpec((HID, OUT_DIM), lambda i: (0, 0)),
        ],
        out_specs=[
            pl.BlockSpec((BN, HID), lambda i: (i, 0)),
            pl.BlockSpec((BN, OUT_DIM), lambda i: (i, 0)),
        ],
        out_shape=[
            jax.ShapeDtypeStruct((NPAD, HID), jnp.float32),
            jax.ShapeDtypeStruct((NPAD, OUT_DIM), jnp.float32),
        ],
    )(p2a, p2b, g1a, g1b, dinv8, W2, b2, W3)


def _fin_body(p_ref, g_ref, dinv_ref, b_ref, h_ref):
    dinv = dinv_ref[:, 0:1]
    h_ref[...] = (p_ref[0] + p_ref[1] + g_ref[...]) * dinv + b_ref[...]


def _fin_call(p3, g3, dinv8, b3):
    return pl.pallas_call(
        _fin_body,
        grid=(GRID,),
        in_specs=[
            pl.BlockSpec((NC, BN, OUT_DIM), lambda i: (0, i, 0)),
            pl.BlockSpec((BN, OUT_DIM), lambda i: (i, 0)),
            pl.BlockSpec((BN, 16), lambda i: (i, 0)),
            pl.BlockSpec((1, OUT_DIM), lambda i: (0, 0)),
        ],
        out_specs=pl.BlockSpec((BN, OUT_DIM), lambda i: (i, 0)),
        out_shape=jax.ShapeDtypeStruct((NPAD, OUT_DIM), jnp.float32),
    )(p3, g3, dinv8, b3)


def kernel(x, edge_index, W1, b1, W2, b2, W3, b3):
    f32 = jnp.float32
    pad = jnp.arange(EP - E, dtype=jnp.int32)
    src = jnp.concatenate(
        [edge_index[0], pad % N]).reshape(NW, NB, B)
    dst = jnp.concatenate(
        [edge_index[1], N + pad % (NPAD - N)]).reshape(NW, NB, B)
    xp = jnp.pad(x, ((0, NPAD - N), (0, 0)))

    init2 = jnp.stack([jnp.ones((RPT, IN_DIM), f32), jnp.zeros((RPT, IN_DIM), f32)])
    ones1 = jnp.ones((B, IN_DIM), f32)
    zeros = jnp.zeros((RPT, IN_DIM), f32)

    degp = _deg_kernel(dst, init2, ones1)
    dinv8, gx = _dinv_call(degp, xp)

    (p1,) = _agg1(gx, src, dst, zeros)
    h1p, g1a, g1b = _mm1_call(p1, gx, dinv8, W1, b1.reshape(1, HID))

    p2a, p2b = _agg2(g1a, g1b, src, dst, zeros)
    h2p, g3 = _mm2_call(p2a, p2b, g1a, g1b, dinv8, W2,
                        b2.reshape(1, HID), W3)

    (p3,) = _agg1(g3, src, dst, zeros)
    h3p = _fin_call(p3, g3, dinv8, b3.reshape(1, OUT_DIM))

    h1, h2, h3 = h1p[:N], h2p[:N], h3p[:N]
    return (h3, (h1, h2, h3))
